# baseline (speedup 1.0000x reference)
import jax
import jax.numpy as jnp
from jax.experimental import pallas as pl
from jax.experimental.pallas import tpu as pltpu

_ALPHA = 0.3
_ROWS = 2048


def _loss_body(x_ref, y_ref, o_ref):
    i = pl.program_id(0)
    d = y_ref[...] - x_ref[...]
    v = jnp.maximum((1.0 - _ALPHA) * d, -_ALPHA * d)
    p = v.reshape(_ROWS // 8, 8, v.shape[-1]).sum(axis=0)

    @pl.when(i == 0)
    def _():
        o_ref[...] = jnp.zeros_like(o_ref)

    o_ref[...] += p


def kernel(x, y):
    n = x.size
    w = x.shape[-1]
    x2 = x.reshape(-1, w)
    y2 = y.reshape(-1, w)
    grid = x2.shape[0] // _ROWS
    partial = pl.pallas_call(
        _loss_body,
        grid=(grid,),
        in_specs=[
            pl.BlockSpec((_ROWS, w), lambda i: (i, 0)),
            pl.BlockSpec((_ROWS, w), lambda i: (i, 0)),
        ],
        out_specs=pl.BlockSpec((8, w), lambda i: (0, 0)),
        out_shape=jax.ShapeDtypeStruct((8, w), jnp.float32),
        compiler_params=pltpu.CompilerParams(
            dimension_semantics=("arbitrary",),
        ),
    )(x2, y2)
    return jnp.sum(partial) / n


# speedup vs baseline: 1.1098x; 1.1098x over previous
import jax
import jax.numpy as jnp
from jax.experimental import pallas as pl
from jax.experimental.pallas import tpu as pltpu

_ALPHA = 0.3
_ROWS = 2048
_CH = 64


def _make_body(n_steps, inv_n, lanes):
    def _loss_body(x_ref, y_ref, o_ref, acc_ref):
        i = pl.program_id(0)

        @pl.when(i == 0)
        def _():
            acc_ref[...] = jnp.zeros_like(acc_ref)

        def chunk(k, acc):
            xs = x_ref[pl.ds(k * _CH, _CH), :]
            ys = y_ref[pl.ds(k * _CH, _CH), :]
            d = ys - xs
            v = jnp.maximum((1.0 - _ALPHA) * d, -_ALPHA * d)
            return acc + v.reshape(_CH // 8, 8, lanes).sum(axis=0)

        acc_ref[...] += jax.lax.fori_loop(
            0, _ROWS // _CH, chunk, jnp.zeros((8, lanes), jnp.float32)
        )

        @pl.when(i == n_steps - 1)
        def _():
            o_ref[0, 0] = jnp.sum(acc_ref[...]) * inv_n

    return _loss_body


def kernel(x, y):
    n = x.size
    w = x.shape[-1]
    x2 = x.reshape(-1, w)
    y2 = y.reshape(-1, w)
    grid = x2.shape[0] // _ROWS
    out = pl.pallas_call(
        _make_body(grid, 1.0 / n, w),
        grid=(grid,),
        in_specs=[
            pl.BlockSpec((_ROWS, w), lambda i: (i, 0)),
            pl.BlockSpec((_ROWS, w), lambda i: (i, 0)),
        ],
        out_specs=pl.BlockSpec(memory_space=pltpu.SMEM),
        out_shape=jax.ShapeDtypeStruct((1, 1), jnp.float32),
        scratch_shapes=[pltpu.VMEM((8, w), jnp.float32)],
        compiler_params=pltpu.CompilerParams(
            dimension_semantics=("arbitrary",),
        ),
    )(x2, y2)
    return out.reshape(())


# speedup vs baseline: 1.1792x; 1.0626x over previous
import jax
import jax.numpy as jnp
from jax.experimental import pallas as pl
from jax.experimental.pallas import tpu as pltpu

_ALPHA = 0.3
_ROWS = 1024
_CH = 64


def _make_body(n_steps, inv_n, lanes):
    def _loss_body(x_ref, y_ref, o_ref, acc_ref):
        i = pl.program_id(0)

        @pl.when(i == 0)
        def _():
            acc_ref[...] = jnp.zeros_like(acc_ref)

        def chunk(k, acc):
            xs = x_ref[pl.ds(k * _CH, _CH), :]
            ys = y_ref[pl.ds(k * _CH, _CH), :]
            d = ys - xs
            v = jnp.maximum((1.0 - _ALPHA) * d, -_ALPHA * d)
            return acc + v.reshape(_CH // 8, 8, lanes).sum(axis=0)

        acc_ref[...] += jax.lax.fori_loop(
            0, _ROWS // _CH, chunk, jnp.zeros((8, lanes), jnp.float32)
        )

        @pl.when(i == n_steps - 1)
        def _():
            o_ref[0, 0] = jnp.sum(acc_ref[...]) * inv_n

    return _loss_body


def kernel(x, y):
    n = x.size
    w = x.shape[-1]
    x2 = x.reshape(-1, w)
    y2 = y.reshape(-1, w)
    grid = x2.shape[0] // _ROWS
    out = pl.pallas_call(
        _make_body(grid, 1.0 / n, w),
        grid=(grid,),
        in_specs=[
            pl.BlockSpec((_ROWS, w), lambda i: (i, 0)),
            pl.BlockSpec((_ROWS, w), lambda i: (i, 0)),
        ],
        out_specs=pl.BlockSpec(memory_space=pltpu.SMEM),
        out_shape=jax.ShapeDtypeStruct((1, 1), jnp.float32),
        scratch_shapes=[pltpu.VMEM((8, w), jnp.float32)],
        compiler_params=pltpu.CompilerParams(
            dimension_semantics=("arbitrary",),
        ),
    )(x2, y2)
    return out.reshape(())
